# Initial kernel scaffold; baseline (speedup 1.0000x reference)
import jax
import jax.numpy as jnp
from jax.experimental import pallas as pl


def kernel(h, t_e2, edge_index1, edge_index2, e1_to_e2, params):
    raise NotImplementedError("write your pallas kernel here")


# speedup vs baseline: 2.8793x; 2.8793x over previous
import functools

import jax
import jax.numpy as jnp
import numpy as np
from jax import lax
from jax.experimental import pallas as pl
from jax.experimental.pallas import tpu as pltpu
from jax.experimental.pallas import tpu_sc as plsc

NW = 32
F32 = jnp.float32
I32 = jnp.int32


def _mesh():
    return plsc.VectorSubcoreMesh(core_axis_name="c", subcore_axis_name="s")


def _wid():
    return lax.axis_index("s") * 2 + lax.axis_index("c")


def _sc_gather(table, idx, chunk):
    B = idx.shape[0]
    Dd = table.shape[1]
    per_w = B // NW
    nch = per_w // chunk
    assert per_w % chunk == 0 and chunk % 8 == 0

    @functools.partial(
        pl.kernel, mesh=_mesh(),
        out_type=jax.ShapeDtypeStruct((B, Dd), table.dtype),
        scratch_types=[pltpu.VMEM((chunk,), I32), pltpu.VMEM((chunk,), I32),
                       pltpu.VMEM((chunk, Dd), table.dtype),
                       pltpu.VMEM((chunk, Dd), table.dtype),
                       pltpu.SemaphoreType.DMA, pltpu.SemaphoreType.DMA,
                       pltpu.SemaphoreType.DMA, pltpu.SemaphoreType.DMA],
        name="sc_gather",
    )
    def k(table_hbm, idx_hbm, out_hbm, i0, i1, r0, r1, g0, g1, o0, o1):
        base = _wid() * per_w
        idx_v = (i0, i1)
        rows_v = (r0, r1)
        gsem = (g0, g1)
        osem = (o0, o1)

        def start_gather(j):
            b = j % 2
            off = base + j * chunk
            pltpu.sync_copy(idx_hbm.at[pl.ds(off, chunk)], idx_v[b])
            return pltpu.async_copy(table_hbm.at[idx_v[b]],
                                    rows_v[b], gsem[b])

        gd = [None, None]
        od = [None, None]
        gd[0] = start_gather(0)
        for j in range(nch):
            b = j % 2
            nb = 1 - b
            if j + 1 < nch:
                if od[nb] is not None:
                    od[nb].wait()
                    od[nb] = None
                gd[nb] = start_gather(j + 1)
            gd[b].wait()
            off = base + j * chunk
            od[b] = pltpu.async_copy(rows_v[b],
                                     out_hbm.at[pl.ds(off, chunk)], osem[b])
        for b in range(2):
            if od[b] is not None:
                od[b].wait()

    return k(table, idx)


def _sc_gather2_add(table_a, idx_a, table_b, idx_b, chunk):
    B = idx_a.shape[0]
    Dd = table_a.shape[1]
    per_w = B // NW
    nch = per_w // chunk
    assert per_w % chunk == 0 and chunk % 8 == 0

    @functools.partial(
        pl.kernel, mesh=_mesh(),
        out_type=jax.ShapeDtypeStruct((B, Dd), F32),
        scratch_types=[pltpu.VMEM((chunk,), I32), pltpu.VMEM((chunk,), I32),
                       pltpu.VMEM((chunk,), I32), pltpu.VMEM((chunk,), I32),
                       pltpu.VMEM((chunk, Dd), F32),
                       pltpu.VMEM((chunk, Dd), F32),
                       pltpu.SemaphoreType.DMA, pltpu.SemaphoreType.DMA,
                       pltpu.SemaphoreType.DMA, pltpu.SemaphoreType.DMA],
        name="sc_gather2_add",
    )
    def k(ta_hbm, ia_hbm, tb_hbm, ib_hbm, out_hbm,
          ia0, ia1, ib0, ib1, r0, r1, g0, g1, o0, o1):
        base = _wid() * per_w
        ia_v = (ia0, ia1)
        ib_v = (ib0, ib1)
        rows_v = (r0, r1)
        gsem = (g0, g1)
        osem = (o0, o1)

        def start_a(j):
            b = j % 2
            off = base + j * chunk
            pltpu.sync_copy(ia_hbm.at[pl.ds(off, chunk)], ia_v[b])
            pltpu.sync_copy(ib_hbm.at[pl.ds(off, chunk)], ib_v[b])
            return pltpu.async_copy(ta_hbm.at[ia_v[b]],
                                    rows_v[b], gsem[b])

        gd = [None, None]
        od = [None, None]
        gd[0] = start_a(0)
        for j in range(nch):
            b = j % 2
            nb = 1 - b
            if j + 1 < nch:
                if od[nb] is not None:
                    od[nb].wait()
                    od[nb] = None
                gd[nb] = start_a(j + 1)
            gd[b].wait()
            pltpu.async_copy(tb_hbm.at[ib_v[b]], rows_v[b], gsem[b],
                             add=True).wait()
            off = base + j * chunk
            od[b] = pltpu.async_copy(rows_v[b],
                                     out_hbm.at[pl.ds(off, chunk)], osem[b])
        for b in range(2):
            if od[b] is not None:
                od[b].wait()

    return k(table_a, idx_a, table_b, idx_b)


def _sc_scatter_add(zeros, msg, idx0, idx1, acc_rows, chunk):
    B, Dd = msg.shape
    per_s = B // 16
    nch = per_s // chunk
    rows_per_sub = acc_rows // 16
    assert acc_rows % 128 == 0 and per_s % chunk == 0

    @functools.partial(
        pl.kernel, mesh=_mesh(),
        out_type=jax.ShapeDtypeStruct((2, acc_rows, Dd), F32),
        scratch_types=[pltpu.VMEM((chunk,), I32),
                       pltpu.VMEM((chunk, Dd), F32),
                       pltpu.VMEM_SHARED((acc_rows, Dd), F32),
                       pltpu.SemaphoreType.DMA],
        name="sc_scatter_add",
    )
    def k(zeros_hbm, m_hbm, i0_hbm, i1_hbm, out_hbm,
          idx_v, rows_v, acc_sh, sem):
        c = lax.axis_index("c")
        s = lax.axis_index("s")
        r0 = s * rows_per_sub
        pltpu.sync_copy(zeros_hbm.at[pl.ds(r0, rows_per_sub)],
                        acc_sh.at[pl.ds(r0, rows_per_sub)])
        plsc.subcore_barrier()
        base = s * per_s
        def body(i_hbm):
            def go(i, carry):
                off = base + i * chunk
                pltpu.sync_copy(i_hbm.at[pl.ds(off, chunk)], idx_v)
                pltpu.sync_copy(m_hbm.at[pl.ds(off, chunk)], rows_v)
                pltpu.sync_copy(rows_v, acc_sh.at[idx_v], add=True)
                return carry
            return go
        @pl.when(c == 0)
        def _():
            lax.fori_loop(0, nch, body(i0_hbm), 0)
        @pl.when(c == 1)
        def _():
            lax.fori_loop(0, nch, body(i1_hbm), 0)
        plsc.subcore_barrier()
        pltpu.sync_copy(acc_sh.at[pl.ds(r0, rows_per_sub)],
                        out_hbm.at[c, pl.ds(r0, rows_per_sub)])

    return k(zeros, msg, idx0, idx1)


def _sc_scatter_rows(t_ref, rows, widx, sidx, chunk):
    B = sidx.shape[0]
    Dd = rows.shape[1]
    per_w = B // NW
    nch = per_w // chunk
    assert per_w % chunk == 0

    @functools.partial(
        pl.kernel, mesh=_mesh(),
        out_type=(),
        scratch_types=[pltpu.VMEM((chunk,), I32),
                       pltpu.VMEM((chunk,), I32),
                       pltpu.VMEM((chunk, Dd), F32),
                       pltpu.SemaphoreType.DMA],
        name="sc_scatter_rows",
    )
    def k(rows_hbm, widx_hbm, sidx_hbm, t_hbm, widx_v, sidx_v, rows_v, sem):
        base = _wid() * per_w
        def body(i, carry):
            off = base + i * chunk
            pltpu.sync_copy(widx_hbm.at[pl.ds(off, chunk)], widx_v)
            pltpu.sync_copy(sidx_hbm.at[pl.ds(off, chunk)], sidx_v)
            pltpu.async_copy(rows_hbm.at[widx_v], rows_v, sem).wait()
            pltpu.sync_copy(rows_v, t_hbm.at[sidx_v])
            return carry
        lax.fori_loop(0, nch, body, 0)

    k(rows, widx, sidx, t_ref)


def _dot(a, b):
    return jnp.dot(a, b, preferred_element_type=F32)


def _k_node_proj(h, wq, bq, wk, bk, wv, bv, bn):
    n = h.shape[0]
    g = n // bn

    def body(h_ref, wq_r, bq_r, wk_r, bk_r, wv_r, bv_r, oq, ok_, ov):
        x = h_ref[...]
        oq[...] = _dot(x, wq_r[...]) + bq_r[...]
        ok_[...] = _dot(x, wk_r[...]) + bk_r[...]
        ov[...] = _dot(x, wv_r[...]) + bv_r[...]

    full = lambda a: pl.BlockSpec(a.shape, lambda i: (0,) * a.ndim)
    blk = pl.BlockSpec((bn, 128), lambda i: (i, 0))
    out = jax.ShapeDtypeStruct((n, 128), F32)
    return pl.pallas_call(
        body, grid=(g,),
        in_specs=[blk, full(wq), full(bq), full(wk), full(bk), full(wv), full(bv)],
        out_specs=[blk, blk, blk],
        out_shape=[out, out, out],
        name="node_proj",
    )(h, wq, bq, wk, bk, wv, bv)


def _k_logits(qd, ks, t_e2, sel4, wg, bg, be):
    e = qd.shape[0]
    g = e // be

    def body(qd_r, ks_r, t_r, sel_r, wg_r, bg_r, o_r):
        qk = qd_r[...] * ks_r[...]
        o_r[...] = _dot(qk, sel_r[...]) + _dot(t_r[...], wg_r[...]) + bg_r[...]

    full = lambda a: pl.BlockSpec(a.shape, lambda i: (0,) * a.ndim)
    blk = pl.BlockSpec((be, 128), lambda i: (i, 0))
    blk8 = pl.BlockSpec((be, 8), lambda i: (i, 0))
    return pl.pallas_call(
        body, grid=(g,),
        in_specs=[blk, blk, blk, full(sel4), full(wg), full(bg)],
        out_specs=blk8,
        out_shape=jax.ShapeDtypeStruct((e, 8), F32),
        name="logits",
    )(qd, ks, t_e2, sel4, wg, bg)


def _k_colmax(x, bl):
    e = x.shape[0]
    g = e // bl

    def body(x_r, o_r):
        m = jnp.max(x_r[...], axis=0)
        @pl.when(pl.program_id(0) == 0)
        def _():
            o_r[...] = m
        @pl.when(pl.program_id(0) > 0)
        def _():
            o_r[...] = jnp.maximum(o_r[...], m)

    return pl.pallas_call(
        body, grid=(g,),
        in_specs=[pl.BlockSpec((bl, 8), lambda i: (i, 0))],
        out_specs=pl.BlockSpec((8,), lambda i: (0,)),
        out_shape=jax.ShapeDtypeStruct((8,), F32),
        name="colmax",
    )(x)


def _k_colsumexp(x, gmax, bl):
    e = x.shape[0]
    g = e // bl

    def body(x_r, m_r, o_r):
        s = jnp.sum(jnp.exp(x_r[...] - m_r[...]), axis=0)
        @pl.when(pl.program_id(0) == 0)
        def _():
            o_r[...] = s
        @pl.when(pl.program_id(0) > 0)
        def _():
            o_r[...] = o_r[...] + s

    return pl.pallas_call(
        body, grid=(g,),
        in_specs=[pl.BlockSpec((bl, 8), lambda i: (i, 0)),
                  pl.BlockSpec((1, 8), lambda i: (0, 0))],
        out_specs=pl.BlockSpec((8,), lambda i: (0,)),
        out_shape=jax.ShapeDtypeStruct((8,), F32),
        name="colsumexp",
    )(x, gmax)


def _k_msg(logits, vs, gmax, selg, be):
    e = logits.shape[0]
    g = e // be

    def body(l_r, vs_r, m_r, selg_r, o_r):
        a = jnp.exp(l_r[...] - m_r[...])
        o_r[...] = _dot(a, selg_r[...]) * vs_r[...]

    full = lambda a: pl.BlockSpec(a.shape, lambda i: (0,) * a.ndim)
    return pl.pallas_call(
        body, grid=(g,),
        in_specs=[pl.BlockSpec((be, 8), lambda i: (i, 0)),
                  pl.BlockSpec((be, 128), lambda i: (i, 0)),
                  full(gmax), full(selg)],
        out_specs=pl.BlockSpec((be, 128), lambda i: (i, 0)),
        out_shape=jax.ShapeDtypeStruct((e, 128), F32),
        name="msg",
    )(logits, vs, gmax, selg)


def _k_node_update(h, agg, wo, bo, wf1, bf1, wf2, bf2,
                   we1s, we1d, wh1s, wh1d, bn):
    n = h.shape[0]
    g = n // bn

    def body(h_r, a_r, wo_r, bo_r, wf1_r, bf1_r, wf2_r, bf2_r,
             we1s_r, we1d_r, wh1s_r, wh1d_r,
             oh, oa1, oa2, ob1, ob2):
        hp = h_r[...] + _dot(a_r[...], wo_r[...]) + bo_r[...]
        ff = jax.nn.silu(_dot(hp, wf1_r[...]) + bf1_r[...])
        oh[...] = hp + _dot(ff, wf2_r[...]) + bf2_r[...]
        oa1[...] = _dot(hp, we1s_r[...])
        oa2[...] = _dot(hp, we1d_r[...])
        ob1[...] = _dot(hp, wh1s_r[...])
        ob2[...] = _dot(hp, wh1d_r[...])

    full = lambda a: pl.BlockSpec(a.shape, lambda i: (0,) * a.ndim)
    blk = pl.BlockSpec((bn, 128), lambda i: (i, 0))
    out = jax.ShapeDtypeStruct((n, 128), F32)
    return pl.pallas_call(
        body, grid=(g,),
        in_specs=[blk, blk] + [full(a) for a in
                  (wo, bo, wf1, bf1, wf2, bf2, we1s, we1d, wh1s, wh1d)],
        out_specs=[blk] * 5,
        out_shape=[out] * 5,
        name="node_update",
    )(h, agg, wo, bo, wf1, bf1, wf2, bf2, we1s, we1d, wh1s, wh1d)


def _k_edge_mlp(t, ga, w1, b1, w2, b2, be):
    e = t.shape[0]
    g = e // be

    def body(t_r, g_r, w1_r, b1_r, w2_r, b2_r, o_r):
        x = g_r[...] + _dot(t_r[...], w1_r[...]) + b1_r[...]
        o_r[...] = t_r[...] + _dot(jax.nn.silu(x), w2_r[...]) + b2_r[...]

    full = lambda a: pl.BlockSpec(a.shape, lambda i: (0,) * a.ndim)
    blk = pl.BlockSpec((be, 128), lambda i: (i, 0))
    return pl.pallas_call(
        body, grid=(g,),
        in_specs=[blk, blk, full(w1), full(b1), full(w2), full(b2)],
        out_specs=blk,
        out_shape=jax.ShapeDtypeStruct((e, 128), F32),
        name="edge_mlp",
    )(t, ga, w1, b1, w2, b2)


_SEL = np.zeros((128, 8), np.float32)
for _d in range(128):
    _SEL[_d, _d // 16] = 1.0


def _layer(h, t_e2, src1, dst1, src2, dst2, e1_to_e2, p):
    n = h.shape[0]
    e2 = t_e2.shape[0]
    e1 = e1_to_e2.shape[0]
    row = lambda b: b.reshape(1, -1)

    sel4 = jnp.asarray(_SEL / 4.0)
    selt = jnp.asarray(_SEL.T)

    hq, hk, hv = _k_node_proj(h, p['Wq'], row(p['bq']), p['Wk'], row(p['bk']),
                              p['Wv'], row(p['bv']), bn=1000)
    qd = _sc_gather(hq, dst2, chunk=400)
    ks = _sc_gather(hk, src2, chunk=400)
    vs = _sc_gather(hv, src2, chunk=400)

    logits = _k_logits(qd, ks, t_e2, sel4, p['Wg'], row(p['bg']), be=2000)
    gmax = _k_colmax(logits, bl=4000)
    gsum = _k_colsumexp(logits, row(gmax), bl=4000)
    selg = selt / gsum[:, None]
    msg = _k_msg(logits, vs, row(gmax), selg, be=2000)

    half = 5056
    acc_rows = 5120
    in0 = dst2 < half
    idx0 = jnp.where(in0, dst2, half)
    idx1 = jnp.where(in0, half, dst2 - half)
    aggs = _sc_scatter_add(jnp.zeros((acc_rows, 128), F32), msg, idx0, idx1,
                           acc_rows, chunk=400)
    agg = jnp.concatenate([aggs[0, :half], aggs[1, :n - half]], axis=0)

    we1 = p['We1']
    wh1 = p['Wh1']
    h2, a1, a2, b1, b2 = _k_node_update(
        h, agg, p['Wo'], row(p['bo']),
        p['Wf1'], row(p['bf1']), p['Wf2'], row(p['bf2']),
        we1[:128], we1[128:256], wh1[:128], wh1[128:256], bn=1000)

    ga = _sc_gather2_add(a1, src2, a2, dst2, chunk=400)
    t_new = _k_edge_mlp(t_e2, ga, we1[256:], row(p['be1']),
                        p['We2'], row(p['be2']), be=2000)

    sub = _sc_gather(t_new, e1_to_e2, chunk=200)
    gb = _sc_gather2_add(b1, src1, b2, dst1, chunk=200)
    rows_full = _k_edge_mlp(sub, gb, wh1[256:], row(p['bh1']),
                            p['Wh2'], row(p['bh2']), be=2000)

    wm = jnp.full((e2,), -1, I32).at[e1_to_e2].max(jnp.arange(e1, dtype=I32))
    windex = wm[e1_to_e2]
    t_ref = jax.new_ref(t_new)
    _sc_scatter_rows(t_ref, rows_full, windex, e1_to_e2, chunk=200)
    t_out = jax.freeze(t_ref)

    return h2, t_out


def kernel(h, t_e2, edge_index1, edge_index2, e1_to_e2, params):
    src2 = edge_index2[0].astype(I32)
    dst2 = edge_index2[1].astype(I32)
    src1 = edge_index1[0].astype(I32)
    dst1 = edge_index1[1].astype(I32)
    e1i = e1_to_e2.astype(I32)
    for p in params:
        h, t_e2 = _layer(h, t_e2, src1, dst1, src2, dst2, e1i, p)
    return h, t_e2


# speedup vs baseline: 3.2513x; 1.1292x over previous
import functools

import jax
import jax.numpy as jnp
import numpy as np
from jax import lax
from jax.experimental import pallas as pl
from jax.experimental.pallas import tpu as pltpu
from jax.experimental.pallas import tpu_sc as plsc

NW = 32
F32 = jnp.float32
I32 = jnp.int32


def _mesh():
    return plsc.VectorSubcoreMesh(core_axis_name="c", subcore_axis_name="s")


def _wid():
    return lax.axis_index("s") * 2 + lax.axis_index("c")


def _sc_gather(table, idx, chunk):
    B = idx.shape[0]
    Dd = table.shape[1]
    per_w = B // NW
    nch = per_w // chunk
    assert per_w % chunk == 0 and chunk % 8 == 0

    @functools.partial(
        pl.kernel, mesh=_mesh(),
        out_type=jax.ShapeDtypeStruct((B, Dd), table.dtype),
        scratch_types=[pltpu.VMEM((chunk,), I32), pltpu.VMEM((chunk,), I32),
                       pltpu.VMEM((chunk, Dd), table.dtype),
                       pltpu.VMEM((chunk, Dd), table.dtype),
                       pltpu.SemaphoreType.DMA, pltpu.SemaphoreType.DMA,
                       pltpu.SemaphoreType.DMA, pltpu.SemaphoreType.DMA],
        name="sc_gather",
    )
    def k(table_hbm, idx_hbm, out_hbm, i0, i1, r0, r1, g0, g1, o0, o1):
        base = _wid() * per_w
        idx_v = (i0, i1)
        rows_v = (r0, r1)
        gsem = (g0, g1)
        osem = (o0, o1)

        def start_gather(j):
            b = j % 2
            off = base + j * chunk
            pltpu.sync_copy(idx_hbm.at[pl.ds(off, chunk)], idx_v[b])
            return pltpu.async_copy(table_hbm.at[idx_v[b]],
                                    rows_v[b], gsem[b])

        gd = [None, None]
        od = [None, None]
        gd[0] = start_gather(0)
        for j in range(nch):
            b = j % 2
            nb = 1 - b
            if j + 1 < nch:
                if od[nb] is not None:
                    od[nb].wait()
                    od[nb] = None
                gd[nb] = start_gather(j + 1)
            gd[b].wait()
            off = base + j * chunk
            od[b] = pltpu.async_copy(rows_v[b],
                                     out_hbm.at[pl.ds(off, chunk)], osem[b])
        for b in range(2):
            if od[b] is not None:
                od[b].wait()

    return k(table, idx)


def _sc_gather2_add(table_a, idx_a, table_b, idx_b, chunk):
    B = idx_a.shape[0]
    Dd = table_a.shape[1]
    per_w = B // NW
    nch = per_w // chunk
    assert per_w % chunk == 0 and chunk % 8 == 0

    @functools.partial(
        pl.kernel, mesh=_mesh(),
        out_type=jax.ShapeDtypeStruct((B, Dd), F32),
        scratch_types=[pltpu.VMEM((chunk,), I32), pltpu.VMEM((chunk,), I32),
                       pltpu.VMEM((chunk,), I32), pltpu.VMEM((chunk,), I32),
                       pltpu.VMEM((chunk, Dd), F32),
                       pltpu.VMEM((chunk, Dd), F32),
                       pltpu.SemaphoreType.DMA, pltpu.SemaphoreType.DMA,
                       pltpu.SemaphoreType.DMA, pltpu.SemaphoreType.DMA],
        name="sc_gather2_add",
    )
    def k(ta_hbm, ia_hbm, tb_hbm, ib_hbm, out_hbm,
          ia0, ia1, ib0, ib1, r0, r1, g0, g1, o0, o1):
        base = _wid() * per_w
        ia_v = (ia0, ia1)
        ib_v = (ib0, ib1)
        rows_v = (r0, r1)
        gsem = (g0, g1)
        osem = (o0, o1)

        def start_a(j):
            b = j % 2
            off = base + j * chunk
            pltpu.sync_copy(ia_hbm.at[pl.ds(off, chunk)], ia_v[b])
            pltpu.sync_copy(ib_hbm.at[pl.ds(off, chunk)], ib_v[b])
            return pltpu.async_copy(ta_hbm.at[ia_v[b]],
                                    rows_v[b], gsem[b])

        gd = [None, None]
        od = [None, None]
        gd[0] = start_a(0)
        for j in range(nch):
            b = j % 2
            nb = 1 - b
            if j + 1 < nch:
                if od[nb] is not None:
                    od[nb].wait()
                    od[nb] = None
                gd[nb] = start_a(j + 1)
            gd[b].wait()
            pltpu.async_copy(tb_hbm.at[ib_v[b]], rows_v[b], gsem[b],
                             add=True).wait()
            off = base + j * chunk
            od[b] = pltpu.async_copy(rows_v[b],
                                     out_hbm.at[pl.ds(off, chunk)], osem[b])
        for b in range(2):
            if od[b] is not None:
                od[b].wait()

    return k(table_a, idx_a, table_b, idx_b)


def _sc_scatter_add(zeros, msg, idx0, idx1, acc_rows, chunk):
    B, Dd = msg.shape
    per_s = B // 16
    nch = per_s // chunk
    rows_per_sub = acc_rows // 16
    assert acc_rows % 128 == 0 and per_s % chunk == 0

    @functools.partial(
        pl.kernel, mesh=_mesh(),
        out_type=jax.ShapeDtypeStruct((2, acc_rows, Dd), F32),
        scratch_types=[pltpu.VMEM((chunk,), I32),
                       pltpu.VMEM((chunk, Dd), F32),
                       pltpu.VMEM_SHARED((acc_rows, Dd), F32),
                       pltpu.SemaphoreType.DMA],
        name="sc_scatter_add",
    )
    def k(zeros_hbm, m_hbm, i0_hbm, i1_hbm, out_hbm,
          idx_v, rows_v, acc_sh, sem):
        c = lax.axis_index("c")
        s = lax.axis_index("s")
        r0 = s * rows_per_sub
        pltpu.sync_copy(zeros_hbm.at[pl.ds(r0, rows_per_sub)],
                        acc_sh.at[pl.ds(r0, rows_per_sub)])
        plsc.subcore_barrier()
        base = s * per_s

        def body(i_hbm):
            def go(i, carry):
                off = base + i * chunk
                pltpu.sync_copy(i_hbm.at[pl.ds(off, chunk)], idx_v)
                pltpu.sync_copy(m_hbm.at[pl.ds(off, chunk)], rows_v)
                pltpu.sync_copy(rows_v, acc_sh.at[idx_v], add=True)
                return carry
            return go
        @pl.when(c == 0)
        def _():
            lax.fori_loop(0, nch, body(i0_hbm), 0)
        @pl.when(c == 1)
        def _():
            lax.fori_loop(0, nch, body(i1_hbm), 0)
        plsc.subcore_barrier()
        pltpu.sync_copy(acc_sh.at[pl.ds(r0, rows_per_sub)],
                        out_hbm.at[c, pl.ds(r0, rows_per_sub)])

    return k(zeros, msg, idx0, idx1)


def _sc_winner(sidx, n_slots, chunk):
    B = sidx.shape[0]
    slots_per = n_slots // NW
    nch = B // chunk
    ng = chunk // 16
    assert n_slots % NW == 0 and B % chunk == 0 and chunk % 16 == 0

    @functools.partial(
        pl.kernel, mesh=_mesh(),
        out_type=jax.ShapeDtypeStruct((n_slots,), I32),
        scratch_types=[pltpu.VMEM((slots_per,), I32),
                       pltpu.VMEM((chunk,), I32), pltpu.VMEM((chunk,), I32),
                       pltpu.SemaphoreType.DMA, pltpu.SemaphoreType.DMA],
        compiler_params=pltpu.CompilerParams(needs_layout_passes=False),
        name="sc_winner",
    )
    def k(sidx_hbm, wm_hbm, wm_v, i0, i1, s0, s1):
        lo = _wid() * slots_per
        ibuf = (i0, i1)
        isem = (s0, s1)
        neg = jnp.full((16,), jnp.iinfo(jnp.int32).min, I32)

        def init(i, carry):
            wm_v[pl.ds(i * 16, 16)] = neg
            return carry
        lax.fori_loop(0, slots_per // 16, init, 0)

        def start_load(j):
            b = j % 2
            return pltpu.async_copy(sidx_hbm.at[pl.ds(j * chunk, chunk)],
                                    ibuf[b], isem[b])

        def process(j, b):
            def group(g, carry):
                idx = ibuf[b][pl.ds(g * 16, 16)]
                val = j * chunk + g * 16 + lax.iota(I32, 16)
                m = (idx >= lo) & (idx < lo + slots_per)
                loc = jnp.where(m, idx - lo, 0)
                cur = plsc.load_gather(wm_v, [loc], mask=m)
                plsc.store_scatter(wm_v, [loc], jnp.maximum(cur, val), mask=m)
                return carry
            lax.fori_loop(0, ng, group, 0)

        d = [None, None]
        d[0] = start_load(0)
        for j in range(nch):
            b = j % 2
            if j + 1 < nch:
                d[1 - b] = start_load(j + 1)
            d[b].wait()
            process(j, b)
        pltpu.sync_copy(wm_v, wm_hbm.at[pl.ds(lo, slots_per)])

    return k(sidx)


def _sc_scatter_rows(t_ref, rows, wm, sidx, chunk):
    B = sidx.shape[0]
    Dd = rows.shape[1]
    per_w = B // NW
    nch = per_w // chunk
    assert per_w % chunk == 0

    @functools.partial(
        pl.kernel, mesh=_mesh(),
        out_type=(),
        scratch_types=[pltpu.VMEM((chunk,), I32),
                       pltpu.VMEM((chunk,), I32),
                       pltpu.VMEM((chunk, Dd), F32),
                       pltpu.SemaphoreType.DMA],
        name="sc_scatter_rows",
    )
    def k(rows_hbm, wm_hbm, sidx_hbm, t_hbm, widx_v, sidx_v, rows_v, sem):
        base = _wid() * per_w
        def body(i, carry):
            off = base + i * chunk
            pltpu.sync_copy(sidx_hbm.at[pl.ds(off, chunk)], sidx_v)
            pltpu.async_copy(wm_hbm.at[sidx_v], widx_v, sem).wait()
            pltpu.async_copy(rows_hbm.at[widx_v], rows_v, sem).wait()
            pltpu.sync_copy(rows_v, t_hbm.at[sidx_v])
            return carry
        lax.fori_loop(0, nch, body, 0)

    k(rows, wm, sidx, t_ref)


def _dot(a, b):
    return jnp.dot(a, b, preferred_element_type=F32)


def _k_node_proj(h, wq, bq, wk, bk, wv, bv, bn):
    n = h.shape[0]
    g = n // bn

    def body(h_ref, wq_r, bq_r, wk_r, bk_r, wv_r, bv_r, oq, ok_, ov):
        x = h_ref[...]
        oq[...] = _dot(x, wq_r[...]) + bq_r[...]
        ok_[...] = _dot(x, wk_r[...]) + bk_r[...]
        ov[...] = _dot(x, wv_r[...]) + bv_r[...]

    full = lambda a: pl.BlockSpec(a.shape, lambda i: (0,) * a.ndim)
    blk = pl.BlockSpec((bn, 128), lambda i: (i, 0))
    out = jax.ShapeDtypeStruct((n, 128), F32)
    return pl.pallas_call(
        body, grid=(g,),
        in_specs=[blk, full(wq), full(bq), full(wk), full(bk), full(wv), full(bv)],
        out_specs=[blk, blk, blk],
        out_shape=[out, out, out],
        name="node_proj",
    )(h, wq, bq, wk, bk, wv, bv)


def _k_logits(qd, ks, t_e2, sel4, wg, bg, be):
    e = qd.shape[0]
    g = e // be

    def body(qd_r, ks_r, t_r, sel_r, wg_r, bg_r, o_r):
        qk = qd_r[...] * ks_r[...]
        o_r[...] = _dot(qk, sel_r[...]) + _dot(t_r[...], wg_r[...]) + bg_r[...]

    full = lambda a: pl.BlockSpec(a.shape, lambda i: (0,) * a.ndim)
    blk = pl.BlockSpec((be, 128), lambda i: (i, 0))
    blk8 = pl.BlockSpec((be, 8), lambda i: (i, 0))
    return pl.pallas_call(
        body, grid=(g,),
        in_specs=[blk, blk, blk, full(sel4), full(wg), full(bg)],
        out_specs=blk8,
        out_shape=jax.ShapeDtypeStruct((e, 8), F32),
        name="logits",
    )(qd, ks, t_e2, sel4, wg, bg)


def _k_colmax(x, bl):
    e = x.shape[0]
    g = e // bl

    def body(x_r, o_r):
        m = jnp.max(x_r[...], axis=0)
        @pl.when(pl.program_id(0) == 0)
        def _():
            o_r[...] = m
        @pl.when(pl.program_id(0) > 0)
        def _():
            o_r[...] = jnp.maximum(o_r[...], m)

    return pl.pallas_call(
        body, grid=(g,),
        in_specs=[pl.BlockSpec((bl, 8), lambda i: (i, 0))],
        out_specs=pl.BlockSpec((8,), lambda i: (0,)),
        out_shape=jax.ShapeDtypeStruct((8,), F32),
        name="colmax",
    )(x)


def _k_colsumexp(x, gmax, bl):
    e = x.shape[0]
    g = e // bl

    def body(x_r, m_r, o_r):
        s = jnp.sum(jnp.exp(x_r[...] - m_r[...]), axis=0)
        @pl.when(pl.program_id(0) == 0)
        def _():
            o_r[...] = s
        @pl.when(pl.program_id(0) > 0)
        def _():
            o_r[...] = o_r[...] + s

    return pl.pallas_call(
        body, grid=(g,),
        in_specs=[pl.BlockSpec((bl, 8), lambda i: (i, 0)),
                  pl.BlockSpec((1, 8), lambda i: (0, 0))],
        out_specs=pl.BlockSpec((8,), lambda i: (0,)),
        out_shape=jax.ShapeDtypeStruct((8,), F32),
        name="colsumexp",
    )(x, gmax)


def _k_msg(logits, vs, gmax, selg, be):
    e = logits.shape[0]
    g = e // be

    def body(l_r, vs_r, m_r, selg_r, o_r):
        a = jnp.exp(l_r[...] - m_r[...])
        o_r[...] = _dot(a, selg_r[...]) * vs_r[...]

    full = lambda a: pl.BlockSpec(a.shape, lambda i: (0,) * a.ndim)
    return pl.pallas_call(
        body, grid=(g,),
        in_specs=[pl.BlockSpec((be, 8), lambda i: (i, 0)),
                  pl.BlockSpec((be, 128), lambda i: (i, 0)),
                  full(gmax), full(selg)],
        out_specs=pl.BlockSpec((be, 128), lambda i: (i, 0)),
        out_shape=jax.ShapeDtypeStruct((e, 128), F32),
        name="msg",
    )(logits, vs, gmax, selg)


def _k_node_update(h, agg, wo, bo, wf1, bf1, wf2, bf2,
                   we1s, we1d, wh1s, wh1d, bn):
    n = h.shape[0]
    g = n // bn

    def body(h_r, a_r, wo_r, bo_r, wf1_r, bf1_r, wf2_r, bf2_r,
             we1s_r, we1d_r, wh1s_r, wh1d_r,
             oh, oa1, oa2, ob1, ob2):
        hp = h_r[...] + _dot(a_r[...], wo_r[...]) + bo_r[...]
        ff = jax.nn.silu(_dot(hp, wf1_r[...]) + bf1_r[...])
        oh[...] = hp + _dot(ff, wf2_r[...]) + bf2_r[...]
        oa1[...] = _dot(hp, we1s_r[...])
        oa2[...] = _dot(hp, we1d_r[...])
        ob1[...] = _dot(hp, wh1s_r[...])
        ob2[...] = _dot(hp, wh1d_r[...])

    full = lambda a: pl.BlockSpec(a.shape, lambda i: (0,) * a.ndim)
    blk = pl.BlockSpec((bn, 128), lambda i: (i, 0))
    out = jax.ShapeDtypeStruct((n, 128), F32)
    return pl.pallas_call(
        body, grid=(g,),
        in_specs=[blk, blk] + [full(a) for a in
                  (wo, bo, wf1, bf1, wf2, bf2, we1s, we1d, wh1s, wh1d)],
        out_specs=[blk] * 5,
        out_shape=[out] * 5,
        name="node_update",
    )(h, agg, wo, bo, wf1, bf1, wf2, bf2, we1s, we1d, wh1s, wh1d)


def _k_edge_mlp(t, ga, w1, b1, w2, b2, be):
    e = t.shape[0]
    g = e // be

    def body(t_r, g_r, w1_r, b1_r, w2_r, b2_r, o_r):
        x = g_r[...] + _dot(t_r[...], w1_r[...]) + b1_r[...]
        o_r[...] = t_r[...] + _dot(jax.nn.silu(x), w2_r[...]) + b2_r[...]

    full = lambda a: pl.BlockSpec(a.shape, lambda i: (0,) * a.ndim)
    blk = pl.BlockSpec((be, 128), lambda i: (i, 0))
    return pl.pallas_call(
        body, grid=(g,),
        in_specs=[blk, blk, full(w1), full(b1), full(w2), full(b2)],
        out_specs=blk,
        out_shape=jax.ShapeDtypeStruct((e, 128), F32),
        name="edge_mlp",
    )(t, ga, w1, b1, w2, b2)


_SEL = np.zeros((128, 8), np.float32)
for _d in range(128):
    _SEL[_d, _d // 16] = 1.0


def _layer(h, t_e2, src1, dst1, src2, dst2, e1_to_e2, p):
    n = h.shape[0]
    e2 = t_e2.shape[0]
    e1 = e1_to_e2.shape[0]
    row = lambda b: b.reshape(1, -1)

    sel4 = jnp.asarray(_SEL / 4.0)
    selt = jnp.asarray(_SEL.T)

    hq, hk, hv = _k_node_proj(h, p['Wq'], row(p['bq']), p['Wk'], row(p['bk']),
                              p['Wv'], row(p['bv']), bn=1000)
    qd = _sc_gather(hq, dst2, chunk=400)
    ks = _sc_gather(hk, src2, chunk=400)
    vs = _sc_gather(hv, src2, chunk=400)

    logits = _k_logits(qd, ks, t_e2, sel4, p['Wg'], row(p['bg']), be=2000)
    gmax = _k_colmax(logits, bl=4000)
    gsum = _k_colsumexp(logits, row(gmax), bl=4000)
    selg = selt / gsum[:, None]
    msg = _k_msg(logits, vs, row(gmax), selg, be=2000)

    half = 5056
    acc_rows = 5120
    in0 = dst2 < half
    idx0 = jnp.where(in0, dst2, half)
    idx1 = jnp.where(in0, half, dst2 - half)
    aggs = _sc_scatter_add(jnp.zeros((acc_rows, 128), F32), msg, idx0, idx1,
                           acc_rows, chunk=400)
    agg = jnp.concatenate([aggs[0, :half], aggs[1, :n - half]], axis=0)

    we1 = p['We1']
    wh1 = p['Wh1']
    h2, a1, a2, b1, b2 = _k_node_update(
        h, agg, p['Wo'], row(p['bo']),
        p['Wf1'], row(p['bf1']), p['Wf2'], row(p['bf2']),
        we1[:128], we1[128:256], wh1[:128], wh1[128:256], bn=1000)

    ga = _sc_gather2_add(a1, src2, a2, dst2, chunk=400)
    t_new = _k_edge_mlp(t_e2, ga, we1[256:], row(p['be1']),
                        p['We2'], row(p['be2']), be=2000)

    sub = _sc_gather(t_new, e1_to_e2, chunk=200)
    gb = _sc_gather2_add(b1, src1, b2, dst1, chunk=200)
    rows_full = _k_edge_mlp(sub, gb, wh1[256:], row(p['bh1']),
                            p['Wh2'], row(p['bh2']), be=2000)

    wm = _sc_winner(e1_to_e2, e2, chunk=4000)
    t_ref = jax.new_ref(t_new)
    _sc_scatter_rows(t_ref, rows_full, wm, e1_to_e2, chunk=200)
    t_out = jax.freeze(t_ref)

    return h2, t_out


def kernel(h, t_e2, edge_index1, edge_index2, e1_to_e2, params):
    src2 = edge_index2[0].astype(I32)
    dst2 = edge_index2[1].astype(I32)
    src1 = edge_index1[0].astype(I32)
    dst1 = edge_index1[1].astype(I32)
    e1i = e1_to_e2.astype(I32)
    for p in params:
        h, t_e2 = _layer(h, t_e2, src1, dst1, src2, dst2, e1i, p)
    return h, t_e2


# speedup vs baseline: 3.5507x; 1.0921x over previous
import functools

import jax
import jax.numpy as jnp
import numpy as np
from jax import lax
from jax.experimental import pallas as pl
from jax.experimental.pallas import tpu as pltpu
from jax.experimental.pallas import tpu_sc as plsc

NW = 32
F32 = jnp.float32
I32 = jnp.int32


def _mesh():
    return plsc.VectorSubcoreMesh(core_axis_name="c", subcore_axis_name="s")


def _wid():
    return lax.axis_index("s") * 2 + lax.axis_index("c")


def _sc_gather(table, idx, chunk):
    B = idx.shape[0]
    Dd = table.shape[1]
    per_w = B // NW
    nch = per_w // chunk
    assert per_w % chunk == 0 and chunk % 8 == 0

    @functools.partial(
        pl.kernel, mesh=_mesh(),
        out_type=jax.ShapeDtypeStruct((B, Dd), table.dtype),
        scratch_types=[pltpu.VMEM((chunk,), I32), pltpu.VMEM((chunk,), I32),
                       pltpu.VMEM((chunk, Dd), table.dtype),
                       pltpu.VMEM((chunk, Dd), table.dtype),
                       pltpu.SemaphoreType.DMA, pltpu.SemaphoreType.DMA,
                       pltpu.SemaphoreType.DMA, pltpu.SemaphoreType.DMA],
        name="sc_gather",
    )
    def k(table_hbm, idx_hbm, out_hbm, i0, i1, r0, r1, g0, g1, o0, o1):
        base = _wid() * per_w
        idx_v = (i0, i1)
        rows_v = (r0, r1)
        gsem = (g0, g1)
        osem = (o0, o1)

        def start_gather(j):
            b = j % 2
            off = base + j * chunk
            pltpu.sync_copy(idx_hbm.at[pl.ds(off, chunk)], idx_v[b])
            return pltpu.async_copy(table_hbm.at[idx_v[b]],
                                    rows_v[b], gsem[b])

        gd = [None, None]
        od = [None, None]
        gd[0] = start_gather(0)
        for j in range(nch):
            b = j % 2
            nb = 1 - b
            if j + 1 < nch:
                if od[nb] is not None:
                    od[nb].wait()
                    od[nb] = None
                gd[nb] = start_gather(j + 1)
            gd[b].wait()
            off = base + j * chunk
            od[b] = pltpu.async_copy(rows_v[b],
                                     out_hbm.at[pl.ds(off, chunk)], osem[b])
        for b in range(2):
            if od[b] is not None:
                od[b].wait()

    return k(table, idx)


def _sc_gather2_add(table_a, idx_a, table_b, idx_b, chunk):
    B = idx_a.shape[0]
    Dd = table_a.shape[1]
    per_w = B // NW
    nch = per_w // chunk
    assert per_w % chunk == 0 and chunk % 8 == 0

    @functools.partial(
        pl.kernel, mesh=_mesh(),
        out_type=jax.ShapeDtypeStruct((B, Dd), F32),
        scratch_types=[pltpu.VMEM((chunk,), I32), pltpu.VMEM((chunk,), I32),
                       pltpu.VMEM((chunk,), I32), pltpu.VMEM((chunk,), I32),
                       pltpu.VMEM((chunk, Dd), F32),
                       pltpu.VMEM((chunk, Dd), F32),
                       pltpu.SemaphoreType.DMA, pltpu.SemaphoreType.DMA,
                       pltpu.SemaphoreType.DMA, pltpu.SemaphoreType.DMA],
        name="sc_gather2_add",
    )
    def k(ta_hbm, ia_hbm, tb_hbm, ib_hbm, out_hbm,
          ia0, ia1, ib0, ib1, r0, r1, g0, g1, o0, o1):
        base = _wid() * per_w
        ia_v = (ia0, ia1)
        ib_v = (ib0, ib1)
        rows_v = (r0, r1)
        gsem = (g0, g1)
        osem = (o0, o1)

        def start_a(j):
            b = j % 2
            off = base + j * chunk
            pltpu.sync_copy(ia_hbm.at[pl.ds(off, chunk)], ia_v[b])
            pltpu.sync_copy(ib_hbm.at[pl.ds(off, chunk)], ib_v[b])
            return pltpu.async_copy(ta_hbm.at[ia_v[b]],
                                    rows_v[b], gsem[b])

        gd = [None, None]
        od = [None, None]
        gd[0] = start_a(0)
        for j in range(nch):
            b = j % 2
            nb = 1 - b
            if j + 1 < nch:
                if od[nb] is not None:
                    od[nb].wait()
                    od[nb] = None
                gd[nb] = start_a(j + 1)
            gd[b].wait()
            pltpu.async_copy(tb_hbm.at[ib_v[b]], rows_v[b], gsem[b],
                             add=True).wait()
            off = base + j * chunk
            od[b] = pltpu.async_copy(rows_v[b],
                                     out_hbm.at[pl.ds(off, chunk)], osem[b])
        for b in range(2):
            if od[b] is not None:
                od[b].wait()

    return k(table_a, idx_a, table_b, idx_b)


def _sc_scatter_add(zeros, msg, idx0, idx1, acc_rows, chunk):
    B, Dd = msg.shape
    per_s = B // 16
    nch = per_s // chunk
    rows_per_sub = acc_rows // 16
    assert acc_rows % 128 == 0 and per_s % chunk == 0

    @functools.partial(
        pl.kernel, mesh=_mesh(),
        out_type=jax.ShapeDtypeStruct((2, acc_rows, Dd), F32),
        scratch_types=[pltpu.VMEM((chunk,), I32), pltpu.VMEM((chunk,), I32),
                       pltpu.VMEM((chunk, Dd), F32),
                       pltpu.VMEM((chunk, Dd), F32),
                       pltpu.VMEM_SHARED((acc_rows, Dd), F32),
                       pltpu.SemaphoreType.DMA, pltpu.SemaphoreType.DMA],
        name="sc_scatter_add",
    )
    def k(zeros_hbm, m_hbm, i0_hbm, i1_hbm, out_hbm,
          iv0, iv1, rv0, rv1, acc_sh, m0, m1):
        c = lax.axis_index("c")
        s = lax.axis_index("s")
        r0 = s * rows_per_sub
        pltpu.sync_copy(zeros_hbm.at[pl.ds(r0, rows_per_sub)],
                        acc_sh.at[pl.ds(r0, rows_per_sub)])
        plsc.subcore_barrier()
        base = s * per_s
        idx_v = (iv0, iv1)
        rows_v = (rv0, rv1)
        msem = (m0, m1)

        def body(i_hbm):
            def start(j):
                b = j % 2
                off = base + j * chunk
                pltpu.sync_copy(i_hbm.at[pl.ds(off, chunk)], idx_v[b])
                return pltpu.async_copy(m_hbm.at[pl.ds(off, chunk)],
                                        rows_v[b], msem[b])
            d = [None, None]
            d[0] = start(0)
            for j in range(nch):
                b = j % 2
                if j + 1 < nch:
                    d[1 - b] = start(j + 1)
                d[b].wait()
                pltpu.sync_copy(rows_v[b], acc_sh.at[idx_v[b]], add=True)
        @pl.when(c == 0)
        def _():
            body(i0_hbm)
        @pl.when(c == 1)
        def _():
            body(i1_hbm)
        plsc.subcore_barrier()
        pltpu.sync_copy(acc_sh.at[pl.ds(r0, rows_per_sub)],
                        out_hbm.at[c, pl.ds(r0, rows_per_sub)])

    return k(zeros, msg, idx0, idx1)


def _sc_winner(sidx, n_slots, chunk):
    B = sidx.shape[0]
    slots_per = n_slots // NW
    nch = B // chunk
    ng = chunk // 16
    assert n_slots % NW == 0 and B % chunk == 0 and chunk % 16 == 0

    @functools.partial(
        pl.kernel, mesh=_mesh(),
        out_type=jax.ShapeDtypeStruct((n_slots,), I32),
        scratch_types=[pltpu.VMEM((slots_per,), I32),
                       pltpu.VMEM((chunk,), I32), pltpu.VMEM((chunk,), I32),
                       pltpu.SemaphoreType.DMA, pltpu.SemaphoreType.DMA],
        compiler_params=pltpu.CompilerParams(needs_layout_passes=False),
        name="sc_winner",
    )
    def k(sidx_hbm, wm_hbm, wm_v, i0, i1, s0, s1):
        lo = _wid() * slots_per
        ibuf = (i0, i1)
        isem = (s0, s1)
        neg = jnp.full((16,), jnp.iinfo(jnp.int32).min, I32)

        def init(i, carry):
            wm_v[pl.ds(i * 16, 16)] = neg
            return carry
        lax.fori_loop(0, slots_per // 16, init, 0)

        def start_load(j):
            b = j % 2
            return pltpu.async_copy(sidx_hbm.at[pl.ds(j * chunk, chunk)],
                                    ibuf[b], isem[b])

        def process(j, b):
            def group(g, carry):
                idx = ibuf[b][pl.ds(g * 16, 16)]
                val = j * chunk + g * 16 + lax.iota(I32, 16)
                m = (idx >= lo) & (idx < lo + slots_per)
                loc = jnp.where(m, idx - lo, 0)
                cur = plsc.load_gather(wm_v, [loc], mask=m)
                plsc.store_scatter(wm_v, [loc], jnp.maximum(cur, val), mask=m)
                return carry
            lax.fori_loop(0, ng, group, 0)

        d = [None, None]
        d[0] = start_load(0)
        for j in range(nch):
            b = j % 2
            if j + 1 < nch:
                d[1 - b] = start_load(j + 1)
            d[b].wait()
            process(j, b)
        pltpu.sync_copy(wm_v, wm_hbm.at[pl.ds(lo, slots_per)])

    return k(sidx)


def _sc_scatter_rows(t_ref, rows, wm, sidx, chunk):
    B = sidx.shape[0]
    Dd = rows.shape[1]
    per_w = B // NW
    nch = per_w // chunk
    assert per_w % chunk == 0

    @functools.partial(
        pl.kernel, mesh=_mesh(),
        out_type=(),
        scratch_types=[pltpu.VMEM((chunk,), I32), pltpu.VMEM((chunk,), I32),
                       pltpu.VMEM((chunk,), I32), pltpu.VMEM((chunk,), I32),
                       pltpu.VMEM((chunk, Dd), F32),
                       pltpu.VMEM((chunk, Dd), F32),
                       pltpu.SemaphoreType.DMA,
                       pltpu.SemaphoreType.DMA, pltpu.SemaphoreType.DMA],
        name="sc_scatter_rows",
    )
    def k(rows_hbm, wm_hbm, sidx_hbm, t_hbm,
          wv0, wv1, sv0, sv1, rv0, rv1, gsem, o0, o1):
        base = _wid() * per_w
        widx_v = (wv0, wv1)
        sidx_v = (sv0, sv1)
        rows_v = (rv0, rv1)
        osem = (o0, o1)

        def chain(j):
            b = j % 2
            off = base + j * chunk
            pltpu.sync_copy(sidx_hbm.at[pl.ds(off, chunk)], sidx_v[b])
            pltpu.async_copy(wm_hbm.at[sidx_v[b]], widx_v[b], gsem).wait()
            pltpu.async_copy(rows_hbm.at[widx_v[b]], rows_v[b], gsem).wait()
            return pltpu.async_copy(rows_v[b], t_hbm.at[sidx_v[b]], osem[b])

        d = [None, None]
        for j in range(nch):
            b = j % 2
            if d[b] is not None:
                d[b].wait()
            d[b] = chain(j)
        for b in range(2):
            if d[b] is not None:
                d[b].wait()

    k(rows, wm, sidx, t_ref)


def _dot(a, b):
    return jnp.dot(a, b, preferred_element_type=F32)


def _pack2(lo, hi):
    lo16 = jax.lax.bitcast_convert_type(lo.astype(jnp.bfloat16), jnp.uint16)
    hi16 = jax.lax.bitcast_convert_type(hi.astype(jnp.bfloat16), jnp.uint16)
    word = (hi16.astype(jnp.uint32) << 16) | lo16.astype(jnp.uint32)
    return jax.lax.bitcast_convert_type(word, I32)


def _unpack_lo(w):
    return jax.lax.bitcast_convert_type(jax.lax.shift_left(w, 16), F32)


def _unpack_hi(w):
    word = jax.lax.bitcast_convert_type(w, jnp.uint32)
    return jax.lax.bitcast_convert_type((word >> 16) << 16, F32)


def _k_node_proj(h, wq, bq, wk, bk, wv, bv, bn):
    n = h.shape[0]
    g = n // bn

    def body(h_ref, wq_r, bq_r, wk_r, bk_r, wv_r, bv_r, oq, okv):
        x = h_ref[...]
        oq[...] = _dot(x, wq_r[...]) + bq_r[...]
        kk = _dot(x, wk_r[...]) + bk_r[...]
        vv = _dot(x, wv_r[...]) + bv_r[...]
        okv[...] = _pack2(kk, vv)

    full = lambda a: pl.BlockSpec(a.shape, lambda i: (0,) * a.ndim)
    blk = pl.BlockSpec((bn, 128), lambda i: (i, 0))
    return pl.pallas_call(
        body, grid=(g,),
        in_specs=[blk, full(wq), full(bq), full(wk), full(bk), full(wv), full(bv)],
        out_specs=[blk, blk],
        out_shape=[jax.ShapeDtypeStruct((n, 128), F32),
                   jax.ShapeDtypeStruct((n, 128), I32)],
        name="node_proj",
    )(h, wq, bq, wk, bk, wv, bv)


def _k_logits(qd, ks, t_e2, sel4, wg, bg, be):
    e = qd.shape[0]
    g = e // be

    def body(qd_r, ks_r, t_r, sel_r, wg_r, bg_r, o_r):
        qk = qd_r[...] * _unpack_lo(ks_r[...])
        o_r[...] = _dot(qk, sel_r[...]) + _dot(t_r[...], wg_r[...]) + bg_r[...]

    full = lambda a: pl.BlockSpec(a.shape, lambda i: (0,) * a.ndim)
    blk = pl.BlockSpec((be, 128), lambda i: (i, 0))
    blk8 = pl.BlockSpec((be, 8), lambda i: (i, 0))
    return pl.pallas_call(
        body, grid=(g,),
        in_specs=[blk, blk, blk, full(sel4), full(wg), full(bg)],
        out_specs=blk8,
        out_shape=jax.ShapeDtypeStruct((e, 8), F32),
        name="logits",
    )(qd, ks, t_e2, sel4, wg, bg)


def _k_colmax(x, bl):
    e = x.shape[0]
    g = e // bl

    def body(x_r, o_r):
        m = jnp.max(x_r[...], axis=0)
        @pl.when(pl.program_id(0) == 0)
        def _():
            o_r[...] = m
        @pl.when(pl.program_id(0) > 0)
        def _():
            o_r[...] = jnp.maximum(o_r[...], m)

    return pl.pallas_call(
        body, grid=(g,),
        in_specs=[pl.BlockSpec((bl, 8), lambda i: (i, 0))],
        out_specs=pl.BlockSpec((8,), lambda i: (0,)),
        out_shape=jax.ShapeDtypeStruct((8,), F32),
        name="colmax",
    )(x)


def _k_colsumexp(x, gmax, bl):
    e = x.shape[0]
    g = e // bl

    def body(x_r, m_r, o_r):
        s = jnp.sum(jnp.exp(x_r[...] - m_r[...]), axis=0)
        @pl.when(pl.program_id(0) == 0)
        def _():
            o_r[...] = s
        @pl.when(pl.program_id(0) > 0)
        def _():
            o_r[...] = o_r[...] + s

    return pl.pallas_call(
        body, grid=(g,),
        in_specs=[pl.BlockSpec((bl, 8), lambda i: (i, 0)),
                  pl.BlockSpec((1, 8), lambda i: (0, 0))],
        out_specs=pl.BlockSpec((8,), lambda i: (0,)),
        out_shape=jax.ShapeDtypeStruct((8,), F32),
        name="colsumexp",
    )(x, gmax)


def _k_msg(logits, vs, gmax, selg, be):
    e = logits.shape[0]
    g = e // be

    def body(l_r, vs_r, m_r, selg_r, o_r):
        a = jnp.exp(l_r[...] - m_r[...])
        o_r[...] = _dot(a, selg_r[...]) * _unpack_hi(vs_r[...])

    full = lambda a: pl.BlockSpec(a.shape, lambda i: (0,) * a.ndim)
    return pl.pallas_call(
        body, grid=(g,),
        in_specs=[pl.BlockSpec((be, 8), lambda i: (i, 0)),
                  pl.BlockSpec((be, 128), lambda i: (i, 0)),
                  full(gmax), full(selg)],
        out_specs=pl.BlockSpec((be, 128), lambda i: (i, 0)),
        out_shape=jax.ShapeDtypeStruct((e, 128), F32),
        name="msg",
    )(logits, vs, gmax, selg)


def _k_node_update(h, agg, wo, bo, wf1, bf1, wf2, bf2,
                   we1s, we1d, wh1s, wh1d, bn):
    n = h.shape[0]
    g = n // bn

    def body(h_r, a_r, wo_r, bo_r, wf1_r, bf1_r, wf2_r, bf2_r,
             we1s_r, we1d_r, wh1s_r, wh1d_r,
             oh, oa1, oa2, ob1, ob2):
        hp = h_r[...] + _dot(a_r[...], wo_r[...]) + bo_r[...]
        ff = jax.nn.silu(_dot(hp, wf1_r[...]) + bf1_r[...])
        oh[...] = hp + _dot(ff, wf2_r[...]) + bf2_r[...]
        oa1[...] = _dot(hp, we1s_r[...])
        oa2[...] = _dot(hp, we1d_r[...])
        ob1[...] = _dot(hp, wh1s_r[...])
        ob2[...] = _dot(hp, wh1d_r[...])

    full = lambda a: pl.BlockSpec(a.shape, lambda i: (0,) * a.ndim)
    blk = pl.BlockSpec((bn, 128), lambda i: (i, 0))
    out = jax.ShapeDtypeStruct((n, 128), F32)
    return pl.pallas_call(
        body, grid=(g,),
        in_specs=[blk, blk] + [full(a) for a in
                  (wo, bo, wf1, bf1, wf2, bf2, we1s, we1d, wh1s, wh1d)],
        out_specs=[blk] * 5,
        out_shape=[out] * 5,
        name="node_update",
    )(h, agg, wo, bo, wf1, bf1, wf2, bf2, we1s, we1d, wh1s, wh1d)


def _k_edge_mlp(t, ga, w1, b1, w2, b2, be):
    e = t.shape[0]
    g = e // be

    def body(t_r, g_r, w1_r, b1_r, w2_r, b2_r, o_r):
        x = g_r[...] + _dot(t_r[...], w1_r[...]) + b1_r[...]
        o_r[...] = t_r[...] + _dot(jax.nn.silu(x), w2_r[...]) + b2_r[...]

    full = lambda a: pl.BlockSpec(a.shape, lambda i: (0,) * a.ndim)
    blk = pl.BlockSpec((be, 128), lambda i: (i, 0))
    return pl.pallas_call(
        body, grid=(g,),
        in_specs=[blk, blk, full(w1), full(b1), full(w2), full(b2)],
        out_specs=blk,
        out_shape=jax.ShapeDtypeStruct((e, 128), F32),
        name="edge_mlp",
    )(t, ga, w1, b1, w2, b2)


_SEL = np.zeros((128, 8), np.float32)
for _d in range(128):
    _SEL[_d, _d // 16] = 1.0


def _layer(h, t_e2, src1, dst1, src2, dst2, e1_to_e2, p):
    n = h.shape[0]
    e2 = t_e2.shape[0]
    e1 = e1_to_e2.shape[0]
    row = lambda b: b.reshape(1, -1)

    sel4 = jnp.asarray(_SEL / 4.0)
    selt = jnp.asarray(_SEL.T)

    hq, kv = _k_node_proj(h, p['Wq'], row(p['bq']), p['Wk'], row(p['bk']),
                          p['Wv'], row(p['bv']), bn=1000)
    qd = _sc_gather(hq, dst2, chunk=400)
    kvs = _sc_gather(kv, src2, chunk=400)

    logits = _k_logits(qd, kvs, t_e2, sel4, p['Wg'], row(p['bg']), be=2000)
    gmax = _k_colmax(logits, bl=4000)
    gsum = _k_colsumexp(logits, row(gmax), bl=4000)
    selg = selt / gsum[:, None]
    msg = _k_msg(logits, kvs, row(gmax), selg, be=2000)

    half = 5056
    acc_rows = 5120
    in0 = dst2 < half
    idx0 = jnp.where(in0, dst2, half)
    idx1 = jnp.where(in0, half, dst2 - half)
    aggs = _sc_scatter_add(jnp.zeros((acc_rows, 128), F32), msg, idx0, idx1,
                           acc_rows, chunk=200)
    agg = jnp.concatenate([aggs[0, :half], aggs[1, :n - half]], axis=0)

    we1 = p['We1']
    wh1 = p['Wh1']
    h2, a1, a2, b1, b2 = _k_node_update(
        h, agg, p['Wo'], row(p['bo']),
        p['Wf1'], row(p['bf1']), p['Wf2'], row(p['bf2']),
        we1[:128], we1[128:256], wh1[:128], wh1[128:256], bn=1000)

    ga = _sc_gather2_add(a1, src2, a2, dst2, chunk=400)
    t_new = _k_edge_mlp(t_e2, ga, we1[256:], row(p['be1']),
                        p['We2'], row(p['be2']), be=2000)

    sub = _sc_gather(t_new, e1_to_e2, chunk=200)
    gb = _sc_gather2_add(b1, src1, b2, dst1, chunk=200)
    rows_full = _k_edge_mlp(sub, gb, wh1[256:], row(p['bh1']),
                            p['Wh2'], row(p['bh2']), be=2000)

    wm = _sc_winner(e1_to_e2, e2, chunk=4000)
    t_ref = jax.new_ref(t_new)
    _sc_scatter_rows(t_ref, rows_full, wm, e1_to_e2, chunk=200)
    t_out = jax.freeze(t_ref)

    return h2, t_out


def kernel(h, t_e2, edge_index1, edge_index2, e1_to_e2, params):
    src2 = edge_index2[0].astype(I32)
    dst2 = edge_index2[1].astype(I32)
    src1 = edge_index1[0].astype(I32)
    dst1 = edge_index1[1].astype(I32)
    e1i = e1_to_e2.astype(I32)
    for p in params:
        h, t_e2 = _layer(h, t_e2, src1, dst1, src2, dst2, e1i, p)
    return h, t_e2


# speedup vs baseline: 3.6420x; 1.0257x over previous
import functools

import jax
import jax.numpy as jnp
import numpy as np
from jax import lax
from jax.experimental import pallas as pl
from jax.experimental.pallas import tpu as pltpu
from jax.experimental.pallas import tpu_sc as plsc

NW = 32
F32 = jnp.float32
I32 = jnp.int32


def _mesh():
    return plsc.VectorSubcoreMesh(core_axis_name="c", subcore_axis_name="s")


def _wid():
    return lax.axis_index("s") * 2 + lax.axis_index("c")


def _sc_gather(table, idx, chunk):
    B = idx.shape[0]
    Dd = table.shape[1]
    per_w = B // NW
    nch = per_w // chunk
    assert per_w % chunk == 0 and chunk % 8 == 0

    @functools.partial(
        pl.kernel, mesh=_mesh(),
        out_type=jax.ShapeDtypeStruct((B, Dd), table.dtype),
        scratch_types=[pltpu.VMEM((chunk,), I32), pltpu.VMEM((chunk,), I32),
                       pltpu.VMEM((chunk, Dd), table.dtype),
                       pltpu.VMEM((chunk, Dd), table.dtype),
                       pltpu.SemaphoreType.DMA, pltpu.SemaphoreType.DMA,
                       pltpu.SemaphoreType.DMA, pltpu.SemaphoreType.DMA],
        name="sc_gather",
    )
    def k(table_hbm, idx_hbm, out_hbm, i0, i1, r0, r1, g0, g1, o0, o1):
        base = _wid() * per_w
        idx_v = (i0, i1)
        rows_v = (r0, r1)
        gsem = (g0, g1)
        osem = (o0, o1)

        def start_gather(j):
            b = j % 2
            off = base + j * chunk
            pltpu.sync_copy(idx_hbm.at[pl.ds(off, chunk)], idx_v[b])
            return pltpu.async_copy(table_hbm.at[idx_v[b]],
                                    rows_v[b], gsem[b])

        gd = [None, None]
        od = [None, None]
        gd[0] = start_gather(0)
        for j in range(nch):
            b = j % 2
            nb = 1 - b
            if j + 1 < nch:
                if od[nb] is not None:
                    od[nb].wait()
                    od[nb] = None
                gd[nb] = start_gather(j + 1)
            gd[b].wait()
            off = base + j * chunk
            od[b] = pltpu.async_copy(rows_v[b],
                                     out_hbm.at[pl.ds(off, chunk)], osem[b])
        for b in range(2):
            if od[b] is not None:
                od[b].wait()

    return k(table, idx)


def _sc_gather2_add(table_a, idx_a, table_b, idx_b, chunk):
    B = idx_a.shape[0]
    Dd = table_a.shape[1]
    per_w = B // NW
    nch = per_w // chunk
    assert per_w % chunk == 0 and chunk % 8 == 0

    @functools.partial(
        pl.kernel, mesh=_mesh(),
        out_type=jax.ShapeDtypeStruct((B, Dd), F32),
        scratch_types=[pltpu.VMEM((chunk,), I32), pltpu.VMEM((chunk,), I32),
                       pltpu.VMEM((chunk,), I32), pltpu.VMEM((chunk,), I32),
                       pltpu.VMEM((chunk,), I32), pltpu.VMEM((chunk,), I32),
                       pltpu.VMEM((chunk, Dd), F32),
                       pltpu.VMEM((chunk, Dd), F32),
                       pltpu.VMEM((chunk, Dd), F32),
                       pltpu.SemaphoreType.DMA, pltpu.SemaphoreType.DMA,
                       pltpu.SemaphoreType.DMA, pltpu.SemaphoreType.DMA,
                       pltpu.SemaphoreType.DMA, pltpu.SemaphoreType.DMA,
                       pltpu.SemaphoreType.DMA, pltpu.SemaphoreType.DMA,
                       pltpu.SemaphoreType.DMA],
        name="sc_gather2_add",
    )
    def k(ta_hbm, ia_hbm, tb_hbm, ib_hbm, out_hbm,
          ia0, ia1, ia2, ib0, ib1, ib2, r0, r1, r2,
          sa0, sa1, sa2, sb0, sb1, sb2, so0, so1, so2):
        base = _wid() * per_w
        ia_v = (ia0, ia1, ia2)
        ib_v = (ib0, ib1, ib2)
        rows_v = (r0, r1, r2)
        asem = (sa0, sa1, sa2)
        bsem = (sb0, sb1, sb2)
        osem = (so0, so1, so2)

        da = [None, None, None]
        db = [None, None, None]
        do = [None, None, None]
        for j in range(nch + 2):
            if j < nch:
                b = j % 3
                if do[b] is not None:
                    do[b].wait()
                    do[b] = None
                off = base + j * chunk
                pltpu.sync_copy(ia_hbm.at[pl.ds(off, chunk)], ia_v[b])
                pltpu.sync_copy(ib_hbm.at[pl.ds(off, chunk)], ib_v[b])
                da[b] = pltpu.async_copy(ta_hbm.at[ia_v[b]], rows_v[b],
                                         asem[b])
            if 1 <= j < nch + 1:
                b1 = (j - 1) % 3
                da[b1].wait()
                db[b1] = pltpu.async_copy(tb_hbm.at[ib_v[b1]], rows_v[b1],
                                          bsem[b1], add=True)
            if j >= 2:
                b2 = (j - 2) % 3
                db[b2].wait()
                off2 = base + (j - 2) * chunk
                do[b2] = pltpu.async_copy(rows_v[b2],
                                          out_hbm.at[pl.ds(off2, chunk)],
                                          osem[b2])
        for b in range(3):
            if do[b] is not None:
                do[b].wait()

    return k(table_a, idx_a, table_b, idx_b)


def _sc_scatter_add(zeros, msg, idx0, idx1, acc_rows, chunk):
    B, Dd = msg.shape
    per_s = B // 16
    nch = per_s // chunk
    rows_per_sub = acc_rows // 16
    assert acc_rows % 128 == 0 and per_s % chunk == 0

    @functools.partial(
        pl.kernel, mesh=_mesh(),
        out_type=jax.ShapeDtypeStruct((2, acc_rows, Dd), F32),
        scratch_types=[pltpu.VMEM((chunk,), I32), pltpu.VMEM((chunk,), I32),
                       pltpu.VMEM((chunk, Dd), F32),
                       pltpu.VMEM((chunk, Dd), F32),
                       pltpu.VMEM_SHARED((acc_rows, Dd), F32),
                       pltpu.SemaphoreType.DMA, pltpu.SemaphoreType.DMA,
                       pltpu.SemaphoreType.DMA, pltpu.SemaphoreType.DMA],
        name="sc_scatter_add",
    )
    def k(zeros_hbm, m_hbm, i0_hbm, i1_hbm, out_hbm,
          iv0, iv1, rv0, rv1, acc_sh, m0, m1, a0, a1):
        c = lax.axis_index("c")
        s = lax.axis_index("s")
        r0 = s * rows_per_sub
        pltpu.sync_copy(zeros_hbm.at[pl.ds(r0, rows_per_sub)],
                        acc_sh.at[pl.ds(r0, rows_per_sub)])
        plsc.subcore_barrier()
        base = s * per_s
        idx_v = (iv0, iv1)
        rows_v = (rv0, rv1)
        msem = (m0, m1)
        asem = (a0, a1)

        def body(i_hbm):
            def start(j):
                b = j % 2
                off = base + j * chunk
                pltpu.sync_copy(i_hbm.at[pl.ds(off, chunk)], idx_v[b])
                return pltpu.async_copy(m_hbm.at[pl.ds(off, chunk)],
                                        rows_v[b], msem[b])
            d = [None, None]
            da = [None, None]
            d[0] = start(0)
            for j in range(nch):
                b = j % 2
                nb = 1 - b
                if j + 1 < nch:
                    if da[nb] is not None:
                        da[nb].wait()
                        da[nb] = None
                    d[nb] = start(j + 1)
                d[b].wait()
                da[b] = pltpu.async_copy(rows_v[b], acc_sh.at[idx_v[b]],
                                         asem[b], add=True)
            for b in range(2):
                if da[b] is not None:
                    da[b].wait()
        @pl.when(c == 0)
        def _():
            body(i0_hbm)
        @pl.when(c == 1)
        def _():
            body(i1_hbm)
        plsc.subcore_barrier()
        pltpu.sync_copy(acc_sh.at[pl.ds(r0, rows_per_sub)],
                        out_hbm.at[c, pl.ds(r0, rows_per_sub)])

    return k(zeros, msg, idx0, idx1)


def _sc_winner(sidx, n_slots, chunk):
    B = sidx.shape[0]
    slots_per = n_slots // NW
    nch = B // chunk
    ng = chunk // 16
    assert n_slots % NW == 0 and B % chunk == 0 and chunk % 16 == 0

    @functools.partial(
        pl.kernel, mesh=_mesh(),
        out_type=jax.ShapeDtypeStruct((n_slots,), I32),
        scratch_types=[pltpu.VMEM((slots_per,), I32),
                       pltpu.VMEM((chunk,), I32), pltpu.VMEM((chunk,), I32),
                       pltpu.SemaphoreType.DMA, pltpu.SemaphoreType.DMA],
        compiler_params=pltpu.CompilerParams(needs_layout_passes=False),
        name="sc_winner",
    )
    def k(sidx_hbm, wm_hbm, wm_v, i0, i1, s0, s1):
        lo = _wid() * slots_per
        ibuf = (i0, i1)
        isem = (s0, s1)
        neg = jnp.full((16,), jnp.iinfo(jnp.int32).min, I32)

        def init(i, carry):
            wm_v[pl.ds(i * 16, 16)] = neg
            return carry
        lax.fori_loop(0, slots_per // 16, init, 0)

        def start_load(j):
            b = j % 2
            return pltpu.async_copy(sidx_hbm.at[pl.ds(j * chunk, chunk)],
                                    ibuf[b], isem[b])

        def process(j, b):
            def group(g, carry):
                idx = ibuf[b][pl.ds(g * 16, 16)]
                val = j * chunk + g * 16 + lax.iota(I32, 16)
                m = (idx >= lo) & (idx < lo + slots_per)
                loc = jnp.where(m, idx - lo, 0)
                cur = plsc.load_gather(wm_v, [loc], mask=m)
                plsc.store_scatter(wm_v, [loc], jnp.maximum(cur, val), mask=m)
                return carry
            lax.fori_loop(0, ng, group, 0)

        d = [None, None]
        d[0] = start_load(0)
        for j in range(nch):
            b = j % 2
            if j + 1 < nch:
                d[1 - b] = start_load(j + 1)
            d[b].wait()
            process(j, b)
        pltpu.sync_copy(wm_v, wm_hbm.at[pl.ds(lo, slots_per)])

    return k(sidx)


def _sc_scatter_rows(t_ref, rows, wm, sidx, chunk):
    B = sidx.shape[0]
    Dd = rows.shape[1]
    per_w = B // NW
    nch = per_w // chunk
    assert per_w % chunk == 0

    @functools.partial(
        pl.kernel, mesh=_mesh(),
        out_type=(),
        scratch_types=[pltpu.VMEM((chunk,), I32), pltpu.VMEM((chunk,), I32),
                       pltpu.VMEM((chunk,), I32), pltpu.VMEM((chunk,), I32),
                       pltpu.VMEM((chunk, Dd), F32),
                       pltpu.VMEM((chunk, Dd), F32),
                       pltpu.SemaphoreType.DMA,
                       pltpu.SemaphoreType.DMA, pltpu.SemaphoreType.DMA],
        name="sc_scatter_rows",
    )
    def k(rows_hbm, wm_hbm, sidx_hbm, t_hbm,
          wv0, wv1, sv0, sv1, rv0, rv1, gsem, o0, o1):
        base = _wid() * per_w
        widx_v = (wv0, wv1)
        sidx_v = (sv0, sv1)
        rows_v = (rv0, rv1)
        osem = (o0, o1)

        def chain(j):
            b = j % 2
            off = base + j * chunk
            pltpu.sync_copy(sidx_hbm.at[pl.ds(off, chunk)], sidx_v[b])
            pltpu.async_copy(wm_hbm.at[sidx_v[b]], widx_v[b], gsem).wait()
            pltpu.async_copy(rows_hbm.at[widx_v[b]], rows_v[b], gsem).wait()
            return pltpu.async_copy(rows_v[b], t_hbm.at[sidx_v[b]], osem[b])

        d = [None, None]
        for j in range(nch):
            b = j % 2
            if d[b] is not None:
                d[b].wait()
            d[b] = chain(j)
        for b in range(2):
            if d[b] is not None:
                d[b].wait()

    k(rows, wm, sidx, t_ref)


def _dot(a, b):
    return jnp.dot(a, b, preferred_element_type=F32)


def _pack2(lo, hi):
    lo16 = jax.lax.bitcast_convert_type(lo.astype(jnp.bfloat16), jnp.uint16)
    hi16 = jax.lax.bitcast_convert_type(hi.astype(jnp.bfloat16), jnp.uint16)
    word = (hi16.astype(jnp.uint32) << 16) | lo16.astype(jnp.uint32)
    return jax.lax.bitcast_convert_type(word, I32)


def _unpack_lo(w):
    return jax.lax.bitcast_convert_type(jax.lax.shift_left(w, 16), F32)


def _unpack_hi(w):
    word = jax.lax.bitcast_convert_type(w, jnp.uint32)
    return jax.lax.bitcast_convert_type((word >> 16) << 16, F32)


def _k_node_proj(h, wq, bq, wk, bk, wv, bv, bn):
    n = h.shape[0]
    g = n // bn

    def body(h_ref, wq_r, bq_r, wk_r, bk_r, wv_r, bv_r, oq, okv):
        x = h_ref[...]
        oq[...] = _dot(x, wq_r[...]) + bq_r[...]
        kk = _dot(x, wk_r[...]) + bk_r[...]
        vv = _dot(x, wv_r[...]) + bv_r[...]
        okv[...] = _pack2(kk, vv)

    full = lambda a: pl.BlockSpec(a.shape, lambda i: (0,) * a.ndim)
    blk = pl.BlockSpec((bn, 128), lambda i: (i, 0))
    return pl.pallas_call(
        body, grid=(g,),
        in_specs=[blk, full(wq), full(bq), full(wk), full(bk), full(wv), full(bv)],
        out_specs=[blk, blk],
        out_shape=[jax.ShapeDtypeStruct((n, 128), F32),
                   jax.ShapeDtypeStruct((n, 128), I32)],
        name="node_proj",
    )(h, wq, bq, wk, bk, wv, bv)


def _k_logits(qd, ks, t_e2, sel4, wg, bg, be):
    e = qd.shape[0]
    g = e // be

    def body(qd_r, ks_r, t_r, sel_r, wg_r, bg_r, o_r):
        qk = qd_r[...] * _unpack_lo(ks_r[...])
        o_r[...] = _dot(qk, sel_r[...]) + _dot(t_r[...], wg_r[...]) + bg_r[...]

    full = lambda a: pl.BlockSpec(a.shape, lambda i: (0,) * a.ndim)
    blk = pl.BlockSpec((be, 128), lambda i: (i, 0))
    blk8 = pl.BlockSpec((be, 8), lambda i: (i, 0))
    return pl.pallas_call(
        body, grid=(g,),
        in_specs=[blk, blk, blk, full(sel4), full(wg), full(bg)],
        out_specs=blk8,
        out_shape=jax.ShapeDtypeStruct((e, 8), F32),
        name="logits",
    )(qd, ks, t_e2, sel4, wg, bg)


def _k_colmax(x, bl):
    e = x.shape[0]
    g = e // bl

    def body(x_r, o_r):
        m = jnp.max(x_r[...], axis=0)
        @pl.when(pl.program_id(0) == 0)
        def _():
            o_r[...] = m
        @pl.when(pl.program_id(0) > 0)
        def _():
            o_r[...] = jnp.maximum(o_r[...], m)

    return pl.pallas_call(
        body, grid=(g,),
        in_specs=[pl.BlockSpec((bl, 8), lambda i: (i, 0))],
        out_specs=pl.BlockSpec((8,), lambda i: (0,)),
        out_shape=jax.ShapeDtypeStruct((8,), F32),
        name="colmax",
    )(x)


def _k_colsumexp(x, gmax, bl):
    e = x.shape[0]
    g = e // bl

    def body(x_r, m_r, o_r):
        s = jnp.sum(jnp.exp(x_r[...] - m_r[...]), axis=0)
        @pl.when(pl.program_id(0) == 0)
        def _():
            o_r[...] = s
        @pl.when(pl.program_id(0) > 0)
        def _():
            o_r[...] = o_r[...] + s

    return pl.pallas_call(
        body, grid=(g,),
        in_specs=[pl.BlockSpec((bl, 8), lambda i: (i, 0)),
                  pl.BlockSpec((1, 8), lambda i: (0, 0))],
        out_specs=pl.BlockSpec((8,), lambda i: (0,)),
        out_shape=jax.ShapeDtypeStruct((8,), F32),
        name="colsumexp",
    )(x, gmax)


def _k_msg(logits, vs, gmax, selg, be):
    e = logits.shape[0]
    g = e // be

    def body(l_r, vs_r, m_r, selg_r, o_r):
        a = jnp.exp(l_r[...] - m_r[...])
        o_r[...] = _dot(a, selg_r[...]) * _unpack_hi(vs_r[...])

    full = lambda a: pl.BlockSpec(a.shape, lambda i: (0,) * a.ndim)
    return pl.pallas_call(
        body, grid=(g,),
        in_specs=[pl.BlockSpec((be, 8), lambda i: (i, 0)),
                  pl.BlockSpec((be, 128), lambda i: (i, 0)),
                  full(gmax), full(selg)],
        out_specs=pl.BlockSpec((be, 128), lambda i: (i, 0)),
        out_shape=jax.ShapeDtypeStruct((e, 128), F32),
        name="msg",
    )(logits, vs, gmax, selg)


def _k_node_update(h, agg, wo, bo, wf1, bf1, wf2, bf2,
                   we1s, we1d, wh1s, wh1d, bn):
    n = h.shape[0]
    g = n // bn

    def body(h_r, a_r, wo_r, bo_r, wf1_r, bf1_r, wf2_r, bf2_r,
             we1s_r, we1d_r, wh1s_r, wh1d_r,
             oh, oa1, oa2, ob1, ob2):
        hp = h_r[...] + _dot(a_r[...], wo_r[...]) + bo_r[...]
        ff = jax.nn.silu(_dot(hp, wf1_r[...]) + bf1_r[...])
        oh[...] = hp + _dot(ff, wf2_r[...]) + bf2_r[...]
        oa1[...] = _dot(hp, we1s_r[...])
        oa2[...] = _dot(hp, we1d_r[...])
        ob1[...] = _dot(hp, wh1s_r[...])
        ob2[...] = _dot(hp, wh1d_r[...])

    full = lambda a: pl.BlockSpec(a.shape, lambda i: (0,) * a.ndim)
    blk = pl.BlockSpec((bn, 128), lambda i: (i, 0))
    out = jax.ShapeDtypeStruct((n, 128), F32)
    return pl.pallas_call(
        body, grid=(g,),
        in_specs=[blk, blk] + [full(a) for a in
                  (wo, bo, wf1, bf1, wf2, bf2, we1s, we1d, wh1s, wh1d)],
        out_specs=[blk] * 5,
        out_shape=[out] * 5,
        name="node_update",
    )(h, agg, wo, bo, wf1, bf1, wf2, bf2, we1s, we1d, wh1s, wh1d)


def _k_edge_mlp(t, ga, w1, b1, w2, b2, be):
    e = t.shape[0]
    g = e // be

    def body(t_r, g_r, w1_r, b1_r, w2_r, b2_r, o_r):
        x = g_r[...] + _dot(t_r[...], w1_r[...]) + b1_r[...]
        o_r[...] = t_r[...] + _dot(jax.nn.silu(x), w2_r[...]) + b2_r[...]

    full = lambda a: pl.BlockSpec(a.shape, lambda i: (0,) * a.ndim)
    blk = pl.BlockSpec((be, 128), lambda i: (i, 0))
    return pl.pallas_call(
        body, grid=(g,),
        in_specs=[blk, blk, full(w1), full(b1), full(w2), full(b2)],
        out_specs=blk,
        out_shape=jax.ShapeDtypeStruct((e, 128), F32),
        name="edge_mlp",
    )(t, ga, w1, b1, w2, b2)


_SEL = np.zeros((128, 8), np.float32)
for _d in range(128):
    _SEL[_d, _d // 16] = 1.0


def _layer(h, t_e2, src1, dst1, src2, dst2, e1_to_e2, p):
    n = h.shape[0]
    e2 = t_e2.shape[0]
    e1 = e1_to_e2.shape[0]
    row = lambda b: b.reshape(1, -1)

    sel4 = jnp.asarray(_SEL / 4.0)
    selt = jnp.asarray(_SEL.T)

    hq, kv = _k_node_proj(h, p['Wq'], row(p['bq']), p['Wk'], row(p['bk']),
                          p['Wv'], row(p['bv']), bn=1000)
    qd = _sc_gather(hq, dst2, chunk=400)
    kvs = _sc_gather(kv, src2, chunk=400)

    logits = _k_logits(qd, kvs, t_e2, sel4, p['Wg'], row(p['bg']), be=2000)
    gmax = _k_colmax(logits, bl=4000)
    gsum = _k_colsumexp(logits, row(gmax), bl=4000)
    selg = selt / gsum[:, None]
    msg = _k_msg(logits, kvs, row(gmax), selg, be=2000)

    half = 5056
    acc_rows = 5120
    in0 = dst2 < half
    idx0 = jnp.where(in0, dst2, half)
    idx1 = jnp.where(in0, half, dst2 - half)
    aggs = _sc_scatter_add(jnp.zeros((acc_rows, 128), F32), msg, idx0, idx1,
                           acc_rows, chunk=200)
    agg = jnp.concatenate([aggs[0, :half], aggs[1, :n - half]], axis=0)

    we1 = p['We1']
    wh1 = p['Wh1']
    h2, a1, a2, b1, b2 = _k_node_update(
        h, agg, p['Wo'], row(p['bo']),
        p['Wf1'], row(p['bf1']), p['Wf2'], row(p['bf2']),
        we1[:128], we1[128:256], wh1[:128], wh1[128:256], bn=1000)

    ga = _sc_gather2_add(a1, src2, a2, dst2, chunk=200)
    t_new = _k_edge_mlp(t_e2, ga, we1[256:], row(p['be1']),
                        p['We2'], row(p['be2']), be=2000)

    sub = _sc_gather(t_new, e1_to_e2, chunk=200)
    gb = _sc_gather2_add(b1, src1, b2, dst1, chunk=200)
    rows_full = _k_edge_mlp(sub, gb, wh1[256:], row(p['bh1']),
                            p['Wh2'], row(p['bh2']), be=2000)

    wm = _sc_winner(e1_to_e2, e2, chunk=8000)
    t_ref = jax.new_ref(t_new)
    _sc_scatter_rows(t_ref, rows_full, wm, e1_to_e2, chunk=200)
    t_out = jax.freeze(t_ref)

    return h2, t_out


def kernel(h, t_e2, edge_index1, edge_index2, e1_to_e2, params):
    src2 = edge_index2[0].astype(I32)
    dst2 = edge_index2[1].astype(I32)
    src1 = edge_index1[0].astype(I32)
    dst1 = edge_index1[1].astype(I32)
    e1i = e1_to_e2.astype(I32)
    for p in params:
        h, t_e2 = _layer(h, t_e2, src1, dst1, src2, dst2, e1i, p)
    return h, t_e2
